# baseline (speedup 1.0000x reference)
import functools
import jax
import jax.numpy as jnp
from jax.experimental import pallas as pl
from jax.experimental.pallas import tpu as pltpu

_HIDDEN = 512
_KEY = 64
_VAL = 64
_HEADS = 4
_BATCH = 8
_ROWS = _BATCH * _HEADS

_NSTREAM = 4
_MB = 5000


def _flash_body(*refs, num_blocks):
    (q_ref, wq_ref, bq_ref) = refs[0:3]
    k_refs = refs[3:3 + _NSTREAM]
    v_refs = refs[3 + _NSTREAM:3 + 2 * _NSTREAM]
    u_refs = refs[3 + 2 * _NSTREAM:3 + 3 * _NSTREAM]
    wo_ref, bo_ref = refs[3 + 3 * _NSTREAM:3 + 3 * _NSTREAM + 2]
    out_ref = refs[3 + 3 * _NSTREAM + 2]
    q32_ref, m_ref, l_ref, acc_ref = refs[3 + 3 * _NSTREAM + 3:]

    i = pl.program_id(0)

    @pl.when(i == 0)
    def _init():
        qs = []
        for h in range(_HEADS):
            qh = jax.lax.dot_general(
                q_ref[...], wq_ref[h],
                (((1,), (1,)), ((), ())),
                preferred_element_type=jnp.float32)
            qs.append(qh + bq_ref[h][None, :])
        q32_ref[...] = jnp.concatenate(qs, axis=0) * (1.0 / (_KEY ** 0.5))
        m_ref[...] = jnp.full((_ROWS, 128), -1e30, jnp.float32)
        l_ref[...] = jnp.zeros((_ROWS, 128), jnp.float32)
        acc_ref[...] = jnp.zeros((_ROWS, _VAL), jnp.float32)

    for j in range(_NSTREAM):
        s = jax.lax.dot_general(
            q32_ref[...], k_refs[j][...],
            (((1,), (1,)), ((), ())),
            preferred_element_type=jnp.float32)
        u = u_refs[j][0]
        s = jnp.where(u > 0.0, s, -1e9)

        m_old = m_ref[...][:, :1]
        s_max = jnp.max(s, axis=1, keepdims=True)
        m_new = jnp.maximum(m_old, s_max)
        p = jnp.exp(s - m_new)
        alpha = jnp.exp(m_old - m_new)
        l_new = l_ref[...][:, :1] * alpha + jnp.sum(p, axis=1, keepdims=True)
        pv = jax.lax.dot_general(
            p, v_refs[j][...],
            (((1,), (0,)), ((), ())),
            preferred_element_type=jnp.float32)
        acc_ref[...] = acc_ref[...] * alpha + pv
        m_ref[...] = jnp.broadcast_to(m_new, (_ROWS, 128))
        l_ref[...] = jnp.broadcast_to(l_new, (_ROWS, 128))

    @pl.when(i == num_blocks - 1)
    def _finish():
        acc = acc_ref[...] / l_ref[...][:, :1]
        out = jnp.zeros((_BATCH, _HIDDEN), jnp.float32) + bo_ref[...]
        for h in range(_HEADS):
            ah = acc[h * _BATCH:(h + 1) * _BATCH]
            out = out + jax.lax.dot_general(
                ah, wo_ref[h],
                (((1,), (1,)), ((), ())),
                preferred_element_type=jnp.float32)
        out_ref[...] = out


def kernel(query, W_q, b_q, mem_keys, memory, usage, W_out, b_out):
    mem_size = mem_keys.shape[0]
    mb = _MB
    ns = _NSTREAM
    num_blocks = mem_size // (mb * ns)

    wq_h = W_q.reshape(_HEADS, _KEY, _HIDDEN)
    bq_h = b_q.reshape(_HEADS, _KEY)
    wo_h = W_out.reshape(_HIDDEN, _HEADS, _VAL).transpose(1, 0, 2)
    bo_2d = b_out.reshape(1, _HIDDEN)
    u_3d = usage.reshape(mem_size // mb, 1, mb)

    body = functools.partial(_flash_body, num_blocks=num_blocks)

    def _k_spec(j):
        return pl.BlockSpec((mb, _KEY), lambda i, j=j: (i * ns + j, 0))

    def _u_spec(j):
        return pl.BlockSpec((1, 1, mb), lambda i, j=j: (i * ns + j, 0, 0))

    in_specs = (
        [pl.BlockSpec((_BATCH, _HIDDEN), lambda i: (0, 0)),
         pl.BlockSpec((_HEADS, _KEY, _HIDDEN), lambda i: (0, 0, 0)),
         pl.BlockSpec((_HEADS, _KEY), lambda i: (0, 0))]
        + [_k_spec(j) for j in range(ns)]
        + [_k_spec(j) for j in range(ns)]
        + [_u_spec(j) for j in range(ns)]
        + [pl.BlockSpec((_HEADS, _HIDDEN, _VAL), lambda i: (0, 0, 0)),
           pl.BlockSpec((1, _HIDDEN), lambda i: (0, 0))]
    )
    operands = (
        [query, wq_h, bq_h]
        + [mem_keys] * ns
        + [memory] * ns
        + [u_3d] * ns
        + [wo_h, bo_2d]
    )

    out = pl.pallas_call(
        body,
        grid=(num_blocks,),
        in_specs=in_specs,
        out_specs=pl.BlockSpec((_BATCH, _HIDDEN), lambda i: (0, 0)),
        out_shape=jax.ShapeDtypeStruct((_BATCH, _HIDDEN), jnp.float32),
        scratch_shapes=[
            pltpu.VMEM((_ROWS, _KEY), jnp.float32),
            pltpu.VMEM((_ROWS, 128), jnp.float32),
            pltpu.VMEM((_ROWS, 128), jnp.float32),
            pltpu.VMEM((_ROWS, _VAL), jnp.float32),
        ],
        compiler_params=pltpu.CompilerParams(
            dimension_semantics=("arbitrary",),
        ),
    )(*operands)
    return out
